# baseline (speedup 1.0000x reference)
import numpy as np
import jax
import jax.numpy as jnp
from jax import lax
from jax.experimental import pallas as pl
from jax.experimental.pallas import tpu as pltpu, tpu_sc as plsc

_JNT = np.array([0, 5, 1, 9, 13, 17, 6, 2, 10, 14, 18, 7, 3, 11, 15, 19, 8, 4, 12, 16, 20])
_PERM = (_JNT[:, None] + np.arange(3)[None, :]).flatten()

_ROWS = 16384
_COLS = 63
_NC = 2
_NS = 16
_NW = _NC * _NS
_EPW = _ROWS * _COLS // _NW
_PERIOD = 1008
_NBLK = _EPW // _PERIOD
_NVEC = _PERIOD // 16
_ZPW = _ROWS // _NW

_P = np.arange(_PERIOD)
_TAB = ((_P // _COLS) * _COLS + _PERM[_P % _COLS]).astype(np.int32)


def _body(in_hbm, tab_hbm, out_hbm, z_hbm, in_v, out_v, tab_v, zero_v):
    wid = lax.axis_index("s") * _NC + lax.axis_index("c")
    base = wid * _EPW
    pltpu.sync_copy(tab_hbm, tab_v)
    pltpu.sync_copy(in_hbm.at[pl.ds(base, _EPW)], in_v)

    def blk(k, carry):
        kb = k * _PERIOD

        def vec(v, carry2):
            o = v * 16
            idx = tab_v[pl.ds(o, 16)] + kb
            out_v[pl.ds(kb + o, 16)] = plsc.load_gather(in_v, [idx])
            return carry2

        return lax.fori_loop(0, _NVEC, vec, carry)

    lax.fori_loop(0, _NBLK, blk, 0)

    z16 = jnp.zeros((16,), jnp.float32)
    for z in range(_ZPW // 16):
        zero_v[pl.ds(z * 16, 16)] = z16

    pltpu.sync_copy(out_v, out_hbm.at[pl.ds(base, _EPW)])
    pltpu.sync_copy(zero_v, z_hbm.at[pl.ds(wid * _ZPW, _ZPW)])


def kernel(inputs):
    in_flat = inputs.reshape(-1)
    tab = jnp.asarray(_TAB)
    mesh = plsc.VectorSubcoreMesh(core_axis_name="c", subcore_axis_name="s")
    out_flat, z_flat = pl.kernel(
        _body,
        mesh=mesh,
        out_type=(
            jax.ShapeDtypeStruct((_ROWS * _COLS,), jnp.float32),
            jax.ShapeDtypeStruct((_ROWS,), jnp.float32),
        ),
        scratch_types=[
            pltpu.VMEM((_EPW,), jnp.float32),
            pltpu.VMEM((_EPW,), jnp.float32),
            pltpu.VMEM((_PERIOD,), jnp.int32),
            pltpu.VMEM((_ZPW,), jnp.float32),
        ],
        compiler_params=pltpu.CompilerParams(needs_layout_passes=False),
    )(in_flat, tab)
    return (out_flat.reshape(_ROWS, _COLS), z_flat.reshape(_ROWS, 1))


# speedup vs baseline: 1.0715x; 1.0715x over previous
import numpy as np
import jax
import jax.numpy as jnp
from jax import lax
from jax.experimental import pallas as pl
from jax.experimental.pallas import tpu as pltpu, tpu_sc as plsc

_JNT = np.array([0, 5, 1, 9, 13, 17, 6, 2, 10, 14, 18, 7, 3, 11, 15, 19, 8, 4, 12, 16, 20])
_PERM = (_JNT[:, None] + np.arange(3)[None, :]).flatten()

_ROWS = 16384
_COLS = 63
_NC = 2
_NS = 16
_NW = _NC * _NS
_EPW = _ROWS * _COLS // _NW
_PERIOD = 1008
_NBLK = _EPW // _PERIOD
_NVEC = _PERIOD // 16
_ZPW = _ROWS // _NW

_P = np.arange(_PERIOD)
_TAB = ((_P // _COLS) * _COLS + _PERM[_P % _COLS]).astype(np.int32)


def _body(in_hbm, tab_hbm, out_hbm, z_hbm, in_v, out_v, tab_v, zero_v):
    wid = lax.axis_index("s") * _NC + lax.axis_index("c")
    base = wid * _EPW
    pltpu.sync_copy(tab_hbm, tab_v)
    pltpu.sync_copy(in_hbm.at[pl.ds(base, _EPW)], in_v)

    _UNROLL = 4
    for v in range(_NVEC):
        o = v * 16
        idx0 = tab_v[pl.ds(o, 16)]

        def stepk(g, idx, o=o):
            kb = g * (_UNROLL * _PERIOD)
            for u in range(_UNROLL):
                out_v[pl.ds(kb + u * _PERIOD + o, 16)] = plsc.load_gather(
                    in_v, [idx + u * _PERIOD]
                )
            return idx + _UNROLL * _PERIOD

        lax.fori_loop(0, _NBLK // _UNROLL, stepk, idx0)

    z16 = jnp.zeros((16,), jnp.float32)
    for z in range(_ZPW // 16):
        zero_v[pl.ds(z * 16, 16)] = z16

    pltpu.sync_copy(out_v, out_hbm.at[pl.ds(base, _EPW)])
    pltpu.sync_copy(zero_v, z_hbm.at[pl.ds(wid * _ZPW, _ZPW)])


def kernel(inputs):
    in_flat = inputs.reshape(-1)
    tab = jnp.asarray(_TAB)
    mesh = plsc.VectorSubcoreMesh(core_axis_name="c", subcore_axis_name="s")
    out_flat, z_flat = pl.kernel(
        _body,
        mesh=mesh,
        out_type=(
            jax.ShapeDtypeStruct((_ROWS * _COLS,), jnp.float32),
            jax.ShapeDtypeStruct((_ROWS,), jnp.float32),
        ),
        scratch_types=[
            pltpu.VMEM((_EPW,), jnp.float32),
            pltpu.VMEM((_EPW,), jnp.float32),
            pltpu.VMEM((_PERIOD,), jnp.int32),
            pltpu.VMEM((_ZPW,), jnp.float32),
        ],
        compiler_params=pltpu.CompilerParams(needs_layout_passes=False),
    )(in_flat, tab)
    return (out_flat.reshape(_ROWS, _COLS), z_flat.reshape(_ROWS, 1))
